# baseline (speedup 1.0000x reference)
import functools

import jax
import jax.numpy as jnp
from jax.experimental import pallas as pl


def _leaky(x):
    return jnp.where(x >= 0, x, 0.01 * x)


def _bf_dot(a, b):
    return jax.lax.dot_general(
        a, b, (((1,), (0,)), ((), ())),
        preferred_element_type=jnp.float32)


def _dot(a, b):
    return _bf_dot(a.astype(jnp.bfloat16), b.astype(jnp.bfloat16))


def _gnn_kernel(obs_ref, adj_ref, w1_ref, b1_ref, w2_ref, b2_ref,
                msgw_ref, msgb_ref, wih_ref, whh_ref, bih_ref, bhh_ref,
                state_ref, *, iterations, row_tile):
    obs = obs_ref[0]
    num_nodes = adj_ref.shape[1]
    h = _leaky(_dot(obs, w1_ref[...]) + b1_ref[...])
    state = _leaky(_dot(h, w2_ref[...]) + b2_ref[...])
    d = state.shape[1]
    for _ in range(iterations):
        s_hi = state.astype(jnp.bfloat16)
        tiles = []
        for t in range(num_nodes // row_tile):
            rows = pl.ds(t * row_tile, row_tile)
            adj_t = adj_ref[pl.ds(0, 1), rows, :][0].astype(jnp.bfloat16)
            tiles.append(_bf_dot(adj_t, s_hi))
        agg = jnp.concatenate(tiles, axis=0)
        m = _leaky(_dot(state, msgw_ref[:d, :]) + _dot(agg, msgw_ref[d:, :])
                   + msgb_ref[...])
        gi = _dot(m, wih_ref[...]) + bih_ref[...]
        gh = _dot(state, whh_ref[...]) + bhh_ref[...]
        i_r, i_z, i_n = gi[:, :d], gi[:, d:2 * d], gi[:, 2 * d:]
        h_r, h_z, h_n = gh[:, :d], gh[:, d:2 * d], gh[:, 2 * d:]
        r = jax.nn.sigmoid(i_r + h_r)
        z = jax.nn.sigmoid(i_z + h_z)
        n = jnp.tanh(i_n + r * h_n)
        state = (1.0 - z) * n + z * state
    state_ref[0] = state


def _heads_kernel(state_ref, headw_ref, headb_ref, regw_ref, regb_ref,
                  regallw_ref, regallb_ref, cls_ref, pred_ref, predall_ref):
    s = state_ref[...]
    cls_ref[...] = _dot(s, headw_ref[...]) + headb_ref[...]
    pred_ref[...] = _dot(s, regw_ref[...]) + regb_ref[...]
    predall_ref[...] = _dot(s, regallw_ref[...]) + regallb_ref[...]


def kernel(node_obs, node_adj, enc_W1, enc_b1, enc_W2, enc_b2, msg_W, msg_b,
           W_ih, W_hh, b_ih, b_hh, head_W, head_b, reg_W, reg_b,
           regall_W, regall_b):
    B, N, F_in = node_obs.shape
    D = enc_W2.shape[1]
    ENC = enc_W1.shape[1]
    NB_CLASSES = head_W.shape[1]
    NB_NODES = regall_W.shape[1]

    row2 = lambda v: v.reshape(1, -1)

    state = pl.pallas_call(
        functools.partial(_gnn_kernel, iterations=3, row_tile=2048),
        grid=(B,),
        in_specs=[
            pl.BlockSpec((1, N, F_in), lambda b: (b, 0, 0)),
            pl.BlockSpec((1, N, N), lambda b: (b, 0, 0)),
            pl.BlockSpec((F_in, ENC), lambda b: (0, 0)),
            pl.BlockSpec((1, ENC), lambda b: (0, 0)),
            pl.BlockSpec((ENC, D), lambda b: (0, 0)),
            pl.BlockSpec((1, D), lambda b: (0, 0)),
            pl.BlockSpec((2 * D, D), lambda b: (0, 0)),
            pl.BlockSpec((1, D), lambda b: (0, 0)),
            pl.BlockSpec((D, 3 * D), lambda b: (0, 0)),
            pl.BlockSpec((D, 3 * D), lambda b: (0, 0)),
            pl.BlockSpec((1, 3 * D), lambda b: (0, 0)),
            pl.BlockSpec((1, 3 * D), lambda b: (0, 0)),
        ],
        out_specs=pl.BlockSpec((1, N, D), lambda b: (b, 0, 0)),
        out_shape=jax.ShapeDtypeStruct((B, N, D), jnp.float32),
    )(node_obs, node_adj, enc_W1, row2(enc_b1), enc_W2, row2(enc_b2),
      msg_W, row2(msg_b), W_ih, W_hh, row2(b_ih), row2(b_hh))

    TILE = 1024
    flat = state.reshape(B * N, D)
    grid = (B * N // TILE,)
    cls, pred, pred_all = pl.pallas_call(
        _heads_kernel,
        grid=grid,
        in_specs=[
            pl.BlockSpec((TILE, D), lambda i: (i, 0)),
            pl.BlockSpec((D, NB_CLASSES), lambda i: (0, 0)),
            pl.BlockSpec((1, NB_CLASSES), lambda i: (0, 0)),
            pl.BlockSpec((D, 1), lambda i: (0, 0)),
            pl.BlockSpec((1, 1), lambda i: (0, 0)),
            pl.BlockSpec((D, NB_NODES), lambda i: (0, 0)),
            pl.BlockSpec((1, NB_NODES), lambda i: (0, 0)),
        ],
        out_specs=[
            pl.BlockSpec((TILE, NB_CLASSES), lambda i: (i, 0)),
            pl.BlockSpec((TILE, 1), lambda i: (i, 0)),
            pl.BlockSpec((TILE, NB_NODES), lambda i: (i, 0)),
        ],
        out_shape=[
            jax.ShapeDtypeStruct((B * N, NB_CLASSES), jnp.float32),
            jax.ShapeDtypeStruct((B * N, 1), jnp.float32),
            jax.ShapeDtypeStruct((B * N, NB_NODES), jnp.float32),
        ],
    )(flat, head_W, row2(head_b), reg_W, row2(reg_b),
      regall_W, row2(regall_b))

    return (cls.reshape(B, N, NB_CLASSES),
            pred.reshape(B, N, 1),
            pred_all.reshape(B, N, NB_NODES))


# speedup vs baseline: 1.2410x; 1.2410x over previous
import functools

import jax
import jax.numpy as jnp
from jax.experimental import pallas as pl


def _leaky(x):
    return jnp.where(x >= 0, x, 0.01 * x)


def _bf_dot(a, b):
    return jax.lax.dot_general(
        a, b, (((1,), (0,)), ((), ())),
        preferred_element_type=jnp.float32)


def _dot(a, b):
    return _bf_dot(a.astype(jnp.bfloat16), b.astype(jnp.bfloat16))


def _gnn_kernel(obs_ref, adj_ref, w1_ref, b1_ref, w2_ref, b2_ref,
                msgw_ref, msgb_ref, wih_ref, whh_ref, bih_ref, bhh_ref,
                state_ref, *, iterations, row_tile):
    obs = obs_ref[0]
    num_nodes = adj_ref.shape[1]
    h = _leaky(_dot(obs, w1_ref[...]) + b1_ref[...])
    state = _leaky(_dot(h, w2_ref[...]) + b2_ref[...])
    d = state.shape[1]
    for _ in range(iterations):
        s_hi = state.astype(jnp.bfloat16)
        tiles = []
        for t in range(num_nodes // row_tile):
            rows = pl.ds(t * row_tile, row_tile)
            adj_t = adj_ref[pl.ds(0, 1), rows, :][0].astype(jnp.bfloat16)
            tiles.append(_bf_dot(adj_t, s_hi))
        agg = jnp.concatenate(tiles, axis=0)
        m = _leaky(_dot(state, msgw_ref[:d, :]) + _dot(agg, msgw_ref[d:, :])
                   + msgb_ref[...])
        gi = _dot(m, wih_ref[...]) + bih_ref[...]
        gh = _dot(state, whh_ref[...]) + bhh_ref[...]
        i_r, i_z, i_n = gi[:, :d], gi[:, d:2 * d], gi[:, 2 * d:]
        h_r, h_z, h_n = gh[:, :d], gh[:, d:2 * d], gh[:, 2 * d:]
        r = jax.nn.sigmoid(i_r + h_r)
        z = jax.nn.sigmoid(i_z + h_z)
        n = jnp.tanh(i_n + r * h_n)
        state = (1.0 - z) * n + z * state
    state_ref[0] = state


def _heads_kernel(state_ref, headw_ref, headb_ref, regw_ref, regb_ref,
                  regallw_ref, regallb_ref, cls_ref, pred_ref, predall_ref):
    s = state_ref[...]
    cls_ref[...] = _dot(s, headw_ref[...]) + headb_ref[...]
    pred_ref[...] = _dot(s, regw_ref[...]) + regb_ref[...]
    predall_ref[...] = _dot(s, regallw_ref[...]) + regallb_ref[...]


def kernel(node_obs, node_adj, enc_W1, enc_b1, enc_W2, enc_b2, msg_W, msg_b,
           W_ih, W_hh, b_ih, b_hh, head_W, head_b, reg_W, reg_b,
           regall_W, regall_b):
    B, N, F_in = node_obs.shape
    D = enc_W2.shape[1]
    ENC = enc_W1.shape[1]
    NB_CLASSES = head_W.shape[1]
    NB_NODES = regall_W.shape[1]

    row2 = lambda v: v.reshape(1, -1)

    state = pl.pallas_call(
        functools.partial(_gnn_kernel, iterations=3, row_tile=1024),
        grid=(B,),
        in_specs=[
            pl.BlockSpec((1, N, F_in), lambda b: (b, 0, 0)),
            pl.BlockSpec((1, N, N), lambda b: (b, 0, 0)),
            pl.BlockSpec((F_in, ENC), lambda b: (0, 0)),
            pl.BlockSpec((1, ENC), lambda b: (0, 0)),
            pl.BlockSpec((ENC, D), lambda b: (0, 0)),
            pl.BlockSpec((1, D), lambda b: (0, 0)),
            pl.BlockSpec((2 * D, D), lambda b: (0, 0)),
            pl.BlockSpec((1, D), lambda b: (0, 0)),
            pl.BlockSpec((D, 3 * D), lambda b: (0, 0)),
            pl.BlockSpec((D, 3 * D), lambda b: (0, 0)),
            pl.BlockSpec((1, 3 * D), lambda b: (0, 0)),
            pl.BlockSpec((1, 3 * D), lambda b: (0, 0)),
        ],
        out_specs=pl.BlockSpec((1, N, D), lambda b: (b, 0, 0)),
        out_shape=jax.ShapeDtypeStruct((B, N, D), jnp.float32),
    )(node_obs, node_adj, enc_W1, row2(enc_b1), enc_W2, row2(enc_b2),
      msg_W, row2(msg_b), W_ih, W_hh, row2(b_ih), row2(b_hh))

    TILE = 512
    flat = state.reshape(B * N, D)
    grid = (B * N // TILE,)
    cls, pred, pred_all = pl.pallas_call(
        _heads_kernel,
        grid=grid,
        in_specs=[
            pl.BlockSpec((TILE, D), lambda i: (i, 0)),
            pl.BlockSpec((D, NB_CLASSES), lambda i: (0, 0)),
            pl.BlockSpec((1, NB_CLASSES), lambda i: (0, 0)),
            pl.BlockSpec((D, 1), lambda i: (0, 0)),
            pl.BlockSpec((1, 1), lambda i: (0, 0)),
            pl.BlockSpec((D, NB_NODES), lambda i: (0, 0)),
            pl.BlockSpec((1, NB_NODES), lambda i: (0, 0)),
        ],
        out_specs=[
            pl.BlockSpec((TILE, NB_CLASSES), lambda i: (i, 0)),
            pl.BlockSpec((TILE, 1), lambda i: (i, 0)),
            pl.BlockSpec((TILE, NB_NODES), lambda i: (i, 0)),
        ],
        out_shape=[
            jax.ShapeDtypeStruct((B * N, NB_CLASSES), jnp.float32),
            jax.ShapeDtypeStruct((B * N, 1), jnp.float32),
            jax.ShapeDtypeStruct((B * N, NB_NODES), jnp.float32),
        ],
    )(flat, head_W, row2(head_b), reg_W, row2(reg_b),
      regall_W, row2(regall_b))

    return (cls.reshape(B, N, NB_CLASSES),
            pred.reshape(B, N, 1),
            pred_all.reshape(B, N, NB_NODES))


# speedup vs baseline: 1.2411x; 1.0001x over previous
import functools

import jax
import jax.numpy as jnp
from jax.experimental import pallas as pl


def _leaky(x):
    return jnp.where(x >= 0, x, 0.01 * x)


def _bf_dot(a, b):
    return jax.lax.dot_general(
        a, b, (((1,), (0,)), ((), ())),
        preferred_element_type=jnp.float32)


def _dot(a, b):
    return _bf_dot(a.astype(jnp.bfloat16), b.astype(jnp.bfloat16))


def _gnn_kernel(obs_ref, adj_ref, w1_ref, b1_ref, w2_ref, b2_ref,
                msgw_ref, msgb_ref, wih_ref, whh_ref, bih_ref, bhh_ref,
                state_ref, *, iterations, row_tile):
    obs = obs_ref[0]
    num_nodes = adj_ref.shape[1]
    h = _leaky(_dot(obs, w1_ref[...]) + b1_ref[...])
    state = _leaky(_dot(h, w2_ref[...]) + b2_ref[...])
    d = state.shape[1]
    for _ in range(iterations):
        s_hi = state.astype(jnp.bfloat16)
        tiles = []
        for t in range(num_nodes // row_tile):
            rows = pl.ds(t * row_tile, row_tile)
            adj_t = adj_ref[pl.ds(0, 1), rows, :][0].astype(jnp.bfloat16)
            tiles.append(_bf_dot(adj_t, s_hi))
        agg = jnp.concatenate(tiles, axis=0)
        m = _leaky(_dot(state, msgw_ref[:d, :]) + _dot(agg, msgw_ref[d:, :])
                   + msgb_ref[...])
        gi = _dot(m, wih_ref[...]) + bih_ref[...]
        gh = _dot(state, whh_ref[...]) + bhh_ref[...]
        i_r, i_z, i_n = gi[:, :d], gi[:, d:2 * d], gi[:, 2 * d:]
        h_r, h_z, h_n = gh[:, :d], gh[:, d:2 * d], gh[:, 2 * d:]
        r = jax.nn.sigmoid(i_r + h_r)
        z = jax.nn.sigmoid(i_z + h_z)
        n = jnp.tanh(i_n + r * h_n)
        state = (1.0 - z) * n + z * state
    state_ref[0] = state


def _heads_kernel(state_ref, headw_ref, headb_ref, regw_ref, regb_ref,
                  regallw_ref, regallb_ref, cls_ref, pred_ref, predall_ref):
    s = state_ref[...]
    cls_ref[...] = _dot(s, headw_ref[...]) + headb_ref[...]
    pred_ref[...] = _dot(s, regw_ref[...]) + regb_ref[...]
    predall_ref[...] = _dot(s, regallw_ref[...]) + regallb_ref[...]


def kernel(node_obs, node_adj, enc_W1, enc_b1, enc_W2, enc_b2, msg_W, msg_b,
           W_ih, W_hh, b_ih, b_hh, head_W, head_b, reg_W, reg_b,
           regall_W, regall_b):
    B, N, F_in = node_obs.shape
    D = enc_W2.shape[1]
    ENC = enc_W1.shape[1]
    NB_CLASSES = head_W.shape[1]
    NB_NODES = regall_W.shape[1]

    row2 = lambda v: v.reshape(1, -1)

    state = pl.pallas_call(
        functools.partial(_gnn_kernel, iterations=3, row_tile=1024),
        grid=(B,),
        in_specs=[
            pl.BlockSpec((1, N, F_in), lambda b: (b, 0, 0)),
            pl.BlockSpec((1, N, N), lambda b: (b, 0, 0)),
            pl.BlockSpec((F_in, ENC), lambda b: (0, 0)),
            pl.BlockSpec((1, ENC), lambda b: (0, 0)),
            pl.BlockSpec((ENC, D), lambda b: (0, 0)),
            pl.BlockSpec((1, D), lambda b: (0, 0)),
            pl.BlockSpec((2 * D, D), lambda b: (0, 0)),
            pl.BlockSpec((1, D), lambda b: (0, 0)),
            pl.BlockSpec((D, 3 * D), lambda b: (0, 0)),
            pl.BlockSpec((D, 3 * D), lambda b: (0, 0)),
            pl.BlockSpec((1, 3 * D), lambda b: (0, 0)),
            pl.BlockSpec((1, 3 * D), lambda b: (0, 0)),
        ],
        out_specs=pl.BlockSpec((1, N, D), lambda b: (b, 0, 0)),
        out_shape=jax.ShapeDtypeStruct((B, N, D), jnp.float32),
    )(node_obs, node_adj, enc_W1, row2(enc_b1), enc_W2, row2(enc_b2),
      msg_W, row2(msg_b), W_ih, W_hh, row2(b_ih), row2(b_hh))

    TILE = 1024
    flat = state.reshape(B * N, D)
    grid = (B * N // TILE,)
    cls, pred, pred_all = pl.pallas_call(
        _heads_kernel,
        grid=grid,
        in_specs=[
            pl.BlockSpec((TILE, D), lambda i: (i, 0)),
            pl.BlockSpec((D, NB_CLASSES), lambda i: (0, 0)),
            pl.BlockSpec((1, NB_CLASSES), lambda i: (0, 0)),
            pl.BlockSpec((D, 1), lambda i: (0, 0)),
            pl.BlockSpec((1, 1), lambda i: (0, 0)),
            pl.BlockSpec((D, NB_NODES), lambda i: (0, 0)),
            pl.BlockSpec((1, NB_NODES), lambda i: (0, 0)),
        ],
        out_specs=[
            pl.BlockSpec((TILE, NB_CLASSES), lambda i: (i, 0)),
            pl.BlockSpec((TILE, 1), lambda i: (i, 0)),
            pl.BlockSpec((TILE, NB_NODES), lambda i: (i, 0)),
        ],
        out_shape=[
            jax.ShapeDtypeStruct((B * N, NB_CLASSES), jnp.float32),
            jax.ShapeDtypeStruct((B * N, 1), jnp.float32),
            jax.ShapeDtypeStruct((B * N, NB_NODES), jnp.float32),
        ],
    )(flat, head_W, row2(head_b), reg_W, row2(reg_b),
      regall_W, row2(regall_b))

    return (cls.reshape(B, N, NB_CLASSES),
            pred.reshape(B, N, 1),
            pred_all.reshape(B, N, NB_NODES))
